# baseline (speedup 1.0000x reference)
import jax
import jax.numpy as jnp
from jax import lax
from jax.experimental import pallas as pl
from jax.experimental.pallas import tpu as pltpu
from jax.experimental.pallas import tpu_sc as plsc

_Q = 100000
_B = 16384
_D = 128
_L = 16
_NC = 2
_NS = 16
_NW = _NC * _NS
_PER_W = _B // _NW
_CH = 128
_NCH = _PER_W // _CH
_MAIN = _PER_W - _L


def _scatter_body(bf_hbm, bl_hbm, idx_hbm, outf_hbm, outl_hbm,
                  idx16_v, rows_v, labs_v, labs2_v, didx_v, eidx_v,
                  sem_r, sem_r2, sem_l, sem_s):
    c = lax.axis_index("c")
    s = lax.axis_index("s")
    wid = s * _NC + c
    base = wid * _PER_W

    _HLF = _PER_W // 2
    cp_rows = pltpu.async_copy(
        bf_hbm.at[pl.ds(base * _D, _HLF * _D)],
        rows_v.at[pl.ds(0, _HLF * _D)], sem_r)
    cp_rows2 = pltpu.async_copy(
        bf_hbm.at[pl.ds((base + _HLF) * _D, _HLF * _D)],
        rows_v.at[pl.ds(_HLF * _D, _HLF * _D)], sem_r2)
    cp_labs = pltpu.async_copy(bl_hbm.at[pl.ds(base, _PER_W)], labs_v, sem_l)

    pltpu.sync_copy(idx_hbm, idx16_v)
    ivec0 = idx16_v[...]
    ivec = ivec0 + base + lax.iota(jnp.int32, 16)
    for k in range(_PER_W // _L):
        d = ivec + (k * _L)
        d = jnp.where(d >= _Q, d - _Q, d)
        didx_v[k // (_CH // _L), pl.ds((k % (_CH // _L)) * _L, _L)] = d
        if k == 0:
            eidx_v[0, :] = d
        if k == _PER_W // _L - 1:
            eidx_v[1, :] = d

    dst = ivec0[0] + base
    dst = jnp.where(dst >= _Q, dst - _Q, dst)
    h = (8 - (dst & 7)) & 7
    no_wrap = dst <= _Q - _PER_W

    cp_rows.wait()
    cp_labs.wait()

    for k in range(_MAIN // _L):
        labs2_v[pl.ds(k * _L, _L)] = labs_v[pl.ds(h + k * _L, _L)]

    @pl.when(no_wrap)
    def _():
        cpf = pltpu.async_copy(
            rows_v.at[pl.ds(0, _HLF * _D)],
            outf_hbm.at[pl.ds(pl.multiple_of(dst * _D, 8), _HLF * _D)],
            sem_s)
        cpl = pltpu.async_copy(
            labs2_v, outl_hbm.at[pl.ds(pl.multiple_of(dst + h, 8), _MAIN)],
            sem_l)
        cp_rows2.wait()
        cpf2 = pltpu.async_copy(
            rows_v.at[pl.ds(_HLF * _D, _HLF * _D)],
            outf_hbm.at[pl.ds(pl.multiple_of((dst + _HLF) * _D, 8),
                              _HLF * _D)], sem_s)
        cpf.wait()
        cpf2.wait()
        cpl.wait()

    @pl.when(jnp.logical_not(no_wrap))
    def _():
        cp_rows2.wait()
        def row_copy(r, _):
            dr = dst + r
            dr = jnp.where(dr >= _Q, dr - _Q, dr)
            pltpu.sync_copy(
                rows_v.at[pl.ds(pl.multiple_of(r * _D, 8), _D)],
                outf_hbm.at[pl.ds(pl.multiple_of(dr * _D, 8), _D)])
            return _
        lax.fori_loop(0, _PER_W, row_copy, 0)
        cps = []
        for j in range(_NCH):
            cps.append(pltpu.async_copy(
                labs_v.at[pl.ds(j * _CH, _CH)], outl_hbm.at[didx_v.at[j]],
                sem_l))
        for cp in cps:
            cp.wait()

    e0l = pltpu.async_copy(
        labs_v.at[pl.ds(0, _L)], outl_hbm.at[eidx_v.at[0]], sem_l)
    e1l = pltpu.async_copy(
        labs_v.at[pl.ds(_PER_W - _L, _L)], outl_hbm.at[eidx_v.at[1]], sem_l)
    e0l.wait()
    e1l.wait()


_scatter_fn = pl.kernel(
    _scatter_body,
    out_type=(),
    mesh=plsc.VectorSubcoreMesh(core_axis_name="c", subcore_axis_name="s"),
    scratch_types=[
        pltpu.VMEM((_L,), jnp.int32),
        pltpu.VMEM((_PER_W * _D,), jnp.float32),
        pltpu.VMEM((_PER_W,), jnp.int32),
        pltpu.VMEM((_MAIN,), jnp.int32),
        pltpu.VMEM((_NCH, _CH), jnp.int32),
        pltpu.VMEM((2, _L), jnp.int32),
        pltpu.SemaphoreType.DMA,
        pltpu.SemaphoreType.DMA,
        pltpu.SemaphoreType.DMA,
        pltpu.SemaphoreType.DMA,
    ],
)


def kernel(batch_features, batch_labels, features, labels, index):
    idx0 = jnp.asarray(index, jnp.int32) % _Q
    idx_arr = jnp.full((_L,), idx0, jnp.int32)
    f_ref = jax.new_ref(features.reshape(-1))
    l_ref = jax.new_ref(labels)
    _scatter_fn(batch_features.reshape(-1), batch_labels, idx_arr,
                f_ref, l_ref)
    return f_ref[...].reshape(_Q, _D), l_ref[...]
